# baseline (speedup 1.0000x reference)
import jax
import jax.numpy as jnp
from jax.experimental import pallas as pl

SEQ_BLOCK = 512


def _add_body(x_ref, pe_ref, o_ref):
    o_ref[...] = x_ref[...] + pe_ref[...]


def kernel(x, pe):
    b, s, d = x.shape
    pe_s = pe[:, :s, :]
    n_seq = s // SEQ_BLOCK
    return pl.pallas_call(
        _add_body,
        grid=(n_seq, b),
        in_specs=[
            pl.BlockSpec((1, SEQ_BLOCK, d), lambda i, j: (j, i, 0)),
            pl.BlockSpec((1, SEQ_BLOCK, d), lambda i, j: (0, i, 0)),
        ],
        out_specs=pl.BlockSpec((1, SEQ_BLOCK, d), lambda i, j: (j, i, 0)),
        out_shape=jax.ShapeDtypeStruct((b, s, d), x.dtype),
    )(x, pe_s)


# speedup vs baseline: 1.1594x; 1.1594x over previous
import jax
import jax.numpy as jnp
from jax.experimental import pallas as pl

SEQ_BLOCK = 2048


def _add_body(x_ref, pe_ref, o_ref):
    o_ref[...] = x_ref[...] + pe_ref[...]


def kernel(x, pe):
    b, s, d = x.shape
    pe_s = pe[:, :s, :]
    n_seq = s // SEQ_BLOCK
    return pl.pallas_call(
        _add_body,
        grid=(n_seq, b),
        in_specs=[
            pl.BlockSpec((1, SEQ_BLOCK, d), lambda i, j: (j, i, 0)),
            pl.BlockSpec((1, SEQ_BLOCK, d), lambda i, j: (0, i, 0)),
        ],
        out_specs=pl.BlockSpec((1, SEQ_BLOCK, d), lambda i, j: (j, i, 0)),
        out_shape=jax.ShapeDtypeStruct((b, s, d), x.dtype),
    )(x, pe_s)
